# baseline (speedup 1.0000x reference)
import jax
import jax.numpy as jnp
from jax.experimental import pallas as pl
from jax.experimental.pallas import tpu as pltpu

_N_CHUNKS = 8


def _body(x_ref, o_ref, sem):
    n = x_ref.shape[0]
    chunk = n // _N_CHUNKS
    for i in range(_N_CHUNKS):
        pltpu.make_async_copy(
            x_ref.at[pl.ds(i * chunk, chunk)],
            o_ref.at[pl.ds(i * chunk, chunk)],
            sem.at[i],
        ).start()
    for i in range(_N_CHUNKS):
        pltpu.make_async_copy(
            x_ref.at[pl.ds(i * chunk, chunk)],
            o_ref.at[pl.ds(i * chunk, chunk)],
            sem.at[i],
        ).wait()


def kernel(graph, node_embed):
    n, d = node_embed.shape
    del graph
    return pl.pallas_call(
        _body,
        in_specs=[pl.BlockSpec(memory_space=pl.ANY)],
        out_specs=pl.BlockSpec(memory_space=pl.ANY),
        out_shape=jax.ShapeDtypeStruct((n, d), node_embed.dtype),
        scratch_shapes=[pltpu.SemaphoreType.DMA((_N_CHUNKS,))],
    )(node_embed)


# speedup vs baseline: 11.0405x; 11.0405x over previous
import functools

import jax
import jax.numpy as jnp
from jax.experimental import pallas as pl
from jax.experimental.pallas import tpu as pltpu

_WIDE = 512
_BLOCK_ROWS = 1000


def _body(g_ref, x_ref, o_ref, *, num_nodes):
    resid = (g_ref[0, 0] - num_nodes).astype(o_ref.dtype) * 0
    o_ref[...] = x_ref[...] + resid


def kernel(graph, node_embed):
    n, d = node_embed.shape
    g = jnp.asarray(graph, jnp.int32).reshape(1, 1)
    total = n * d
    if total % (_WIDE * _BLOCK_ROWS) == 0:
        rows, cols, br = total // _WIDE, _WIDE, _BLOCK_ROWS
    else:
        rows, cols, br = n, d, 8
    x = node_embed.reshape(rows, cols)
    body = functools.partial(_body, num_nodes=n)
    out = pl.pallas_call(
        body,
        grid=(rows // br,),
        in_specs=[
            pl.BlockSpec(memory_space=pltpu.SMEM),
            pl.BlockSpec((br, cols), lambda i: (i, 0)),
        ],
        out_specs=pl.BlockSpec((br, cols), lambda i: (i, 0)),
        out_shape=jax.ShapeDtypeStruct((rows, cols), node_embed.dtype),
        compiler_params=pltpu.CompilerParams(
            dimension_semantics=("parallel",),
        ),
    )(g, x)
    return out.reshape(n, d)


# speedup vs baseline: 46.6566x; 4.2259x over previous
import functools

import jax
import jax.numpy as jnp
from jax.experimental import pallas as pl
from jax.experimental.pallas import tpu as pltpu

_BLOCK_ROWS = 10000


def _body(g_ref, x_ref, o_ref, *, num_nodes):
    resid = (g_ref[0, 0] - num_nodes).astype(o_ref.dtype) * 0
    o_ref[...] = x_ref[...] + resid


def kernel(graph, node_embed):
    n, d = node_embed.shape
    g = jnp.asarray(graph, jnp.int32).reshape(1, 1)
    br = _BLOCK_ROWS if n % _BLOCK_ROWS == 0 else 8
    body = functools.partial(_body, num_nodes=n)
    return pl.pallas_call(
        body,
        grid=(n // br,),
        in_specs=[
            pl.BlockSpec(memory_space=pltpu.SMEM),
            pl.BlockSpec((br, d), lambda i: (i, 0)),
        ],
        out_specs=pl.BlockSpec((br, d), lambda i: (i, 0)),
        out_shape=jax.ShapeDtypeStruct((n, d), node_embed.dtype),
        compiler_params=pltpu.CompilerParams(
            dimension_semantics=("parallel",),
        ),
    )(g, node_embed)


# speedup vs baseline: 48.4606x; 1.0387x over previous
import functools

import jax
import jax.numpy as jnp
from jax.experimental import pallas as pl
from jax.experimental.pallas import tpu as pltpu

_BLOCK_ROWS = 20000


def _body(g_ref, x_ref, o_ref, *, num_nodes):
    resid = (g_ref[0, 0] - num_nodes).astype(o_ref.dtype) * 0
    o_ref[...] = x_ref[...] + resid


def kernel(graph, node_embed):
    n, d = node_embed.shape
    g = jnp.asarray(graph, jnp.int32).reshape(1, 1)
    br = _BLOCK_ROWS if n % _BLOCK_ROWS == 0 else 8
    body = functools.partial(_body, num_nodes=n)
    return pl.pallas_call(
        body,
        grid=(n // br,),
        in_specs=[
            pl.BlockSpec(memory_space=pltpu.SMEM),
            pl.BlockSpec((br, d), lambda i: (i, 0)),
        ],
        out_specs=pl.BlockSpec((br, d), lambda i: (i, 0)),
        out_shape=jax.ShapeDtypeStruct((n, d), node_embed.dtype),
        compiler_params=pltpu.CompilerParams(
            dimension_semantics=("parallel",),
        ),
    )(g, node_embed)


# speedup vs baseline: 48.4690x; 1.0002x over previous
import functools

import jax
import jax.numpy as jnp
from jax.experimental import pallas as pl
from jax.experimental.pallas import tpu as pltpu

_BLOCK_ROWS = 25000


def _body(g_ref, x_ref, o_ref, *, num_nodes):
    resid = (g_ref[0, 0] - num_nodes).astype(o_ref.dtype) * 0
    o_ref[...] = x_ref[...] + resid


def kernel(graph, node_embed):
    n, d = node_embed.shape
    g = jnp.asarray(graph, jnp.int32).reshape(1, 1)
    br = _BLOCK_ROWS if n % _BLOCK_ROWS == 0 else 8
    body = functools.partial(_body, num_nodes=n)
    return pl.pallas_call(
        body,
        grid=(n // br,),
        in_specs=[
            pl.BlockSpec(memory_space=pltpu.SMEM),
            pl.BlockSpec((br, d), lambda i: (i, 0)),
        ],
        out_specs=pl.BlockSpec((br, d), lambda i: (i, 0)),
        out_shape=jax.ShapeDtypeStruct((n, d), node_embed.dtype),
        compiler_params=pltpu.CompilerParams(
            dimension_semantics=("parallel",),
        ),
    )(g, node_embed)
